# baseline (speedup 1.0000x reference)
import functools

import jax
import jax.numpy as jnp
from jax import lax
from jax.experimental import pallas as pl
from jax.experimental.pallas import tpu as pltpu
from jax.experimental.pallas import tpu_sc as plsc

NN = 10000
EE = 320000
DF = 128
H = 32
NG = 128
NL = 10

NC = 2
NS = 16
NW = NC * NS
CH = 128
KCH = 79
EP = NW * KCH * CH
NP = NN + 112
RT = NP // NS

_mesh = plsc.VectorSubcoreMesh(core_axis_name="c", subcore_axis_name="s")
_sc_params = pltpu.CompilerParams(use_tc_tiling_on_sc=False)


EPT = EE // NW
EFILL = KCH * CH - EPT


def _stage_edges(e_h, wid, src_v, dst_v):
    pltpu.sync_copy(e_h.at[0, pl.ds(wid * EPT, EPT)], src_v.at[pl.ds(0, EPT)])
    pltpu.sync_copy(e_h.at[1, pl.ds(wid * EPT, EPT)], dst_v.at[pl.ds(0, EPT)])
    pad16 = lax.iota(jnp.int32, 16) + NN

    def fill(g, carry):
        src_v[pl.ds(EPT + g * 16, 16)] = pad16 + g * 16
        dst_v[pl.ds(EPT + g * 16, 16)] = pad16 + g * 16
        return carry

    lax.fori_loop(0, EFILL // 16, fill, 0)


def _deg_body(e_h, ones_h, z_h, deg_h,
              src_v, dst_v, ones_v, acc_in, acc_out, isem, osem):
    c = lax.axis_index("c")
    s = lax.axis_index("s")
    wid = s * NC + c
    pltpu.sync_copy(z_h.at[pl.ds(s * RT, RT)], acc_in.at[pl.ds(s * RT, RT)])
    pltpu.sync_copy(z_h.at[pl.ds(s * RT, RT)], acc_out.at[pl.ds(s * RT, RT)])
    pltpu.sync_copy(ones_h, ones_v)
    _stage_edges(e_h, wid, src_v, dst_v)
    plsc.subcore_barrier()

    def body(j, carry):
        pltpu.async_copy(ones_v, acc_in.at[dst_v.at[pl.ds(j * CH, CH)]],
                         isem, add=True)
        pltpu.async_copy(ones_v, acc_out.at[src_v.at[pl.ds(j * CH, CH)]],
                         osem, add=True)
        return carry

    lax.fori_loop(0, KCH, body, 0)

    def drain(j, carry):
        pltpu.make_async_copy(ones_v, acc_in.at[dst_v.at[pl.ds(j * CH, CH)]],
                              isem).wait()
        pltpu.make_async_copy(ones_v, acc_out.at[src_v.at[pl.ds(j * CH, CH)]],
                              osem).wait()
        return carry

    lax.fori_loop(0, KCH, drain, 0)
    plsc.subcore_barrier()
    pltpu.sync_copy(acc_in.at[pl.ds(s * RT, RT)],
                    deg_h.at[pl.ds(s * RT, RT), pl.ds(c * 16, 16)])
    pltpu.sync_copy(acc_out.at[pl.ds(s * RT, RT)],
                    deg_h.at[pl.ds(s * RT, RT), pl.ds(32 + c * 16, 16)])


_sc_deg = pl.kernel(
    _deg_body,
    out_type=jax.ShapeDtypeStruct((NP, 128), jnp.float32),
    mesh=_mesh,
    scratch_types=[
        pltpu.VMEM((KCH * CH,), jnp.int32),
        pltpu.VMEM((KCH * CH,), jnp.int32),
        pltpu.VMEM((CH, 16), jnp.float32),
        pltpu.VMEM_SHARED((NP, 16), jnp.float32),
        pltpu.VMEM_SHARED((NP, 16), jnp.float32),
        pltpu.SemaphoreType.DMA,
        pltpu.SemaphoreType.DMA,
    ],
    compiler_params=_sc_params,
)


NBUF = 4


def _agg_body(p_h, e_h, z_h, out_h,
              src_v, dst_v, rows, acc, gsems, ssems):
    c = lax.axis_index("c")
    s = lax.axis_index("s")
    wid = s * NC + c
    pltpu.sync_copy(z_h.at[pl.ds(s * RT, RT)], acc.at[pl.ds(s * RT, RT)])
    _stage_edges(e_h, wid, src_v, dst_v)
    plsc.subcore_barrier()

    def gsrc(j):
        return p_h.at[src_v.at[pl.ds(j * CH, CH)]]

    def sidx(j):
        return dst_v.at[pl.ds(j * CH, CH)]

    def step(j, b):
        buf = rows.at[b]
        pltpu.make_async_copy(gsrc(j), buf, gsems.at[b]).wait()
        pltpu.async_copy(buf, acc.at[sidx(j)], ssems.at[b], add=True)

        @pl.when(j + 2 < KCH)
        def _():
            b2 = (b + 2) % NBUF
            buf2 = rows.at[b2]

            @pl.when(j >= 2)
            def _():
                pltpu.make_async_copy(
                    buf2, acc.at[sidx(j - 2)], ssems.at[b2]).wait()

            pltpu.async_copy(gsrc(j + 2), buf2, gsems.at[b2])

    pltpu.async_copy(gsrc(0), rows.at[0], gsems.at[0])
    pltpu.async_copy(gsrc(1), rows.at[1], gsems.at[1])

    def body(j, carry):
        for b in range(NBUF):
            @pl.when(lax.rem(j, NBUF) == b)
            def _(j=j, b=b):
                step(j, b)
        return carry

    lax.fori_loop(0, KCH, body, 0)
    for t in range(KCH - NBUF, KCH):
        b = t % NBUF
        pltpu.make_async_copy(rows.at[b], acc.at[sidx(t)], ssems.at[b]).wait()
    plsc.subcore_barrier()
    pltpu.sync_copy(acc.at[pl.ds(s * RT, RT)],
                    out_h.at[pl.ds(s * RT, RT), pl.ds(c * H, H)])


_sc_agg = pl.kernel(
    _agg_body,
    out_type=jax.ShapeDtypeStruct((NP, 128), jnp.float32),
    mesh=_mesh,
    scratch_types=[
        pltpu.VMEM((KCH * CH,), jnp.int32),
        pltpu.VMEM((KCH * CH,), jnp.int32),
        pltpu.VMEM((NBUF, CH, H), jnp.float32),
        pltpu.VMEM_SHARED((NP, H), jnp.float32),
        pltpu.SemaphoreType.DMA((NBUF,)),
        pltpu.SemaphoreType.DMA((NBUF,)),
    ],
    compiler_params=_sc_params,
)


def _prep_body(x_ref, deg_ref, wg_ref, ws_ref,
               p_ref, s_ref, dib_ref, dob_ref):
    deg_in = deg_ref[:, 0:1] + deg_ref[:, 16:17]
    deg_out = deg_ref[:, 32:33] + deg_ref[:, 48:49]
    di = jnp.where(deg_in > 0, lax.rsqrt(jnp.maximum(deg_in, 1.0)), 0.0)
    do = jnp.where(deg_out > 0, lax.rsqrt(jnp.maximum(deg_out, 1.0)), 0.0)
    x = x_ref[...]
    p_ref[...] = do * jnp.dot(x, wg_ref[...], preferred_element_type=jnp.float32)
    s_ref[...] = jnp.dot(x, ws_ref[...], preferred_element_type=jnp.float32)
    dib_ref[...] = jnp.broadcast_to(di, (NP, H))
    dob_ref[...] = jnp.broadcast_to(do, (NP, H))


_tc_prep = pl.pallas_call(
    _prep_body,
    out_shape=(
        jax.ShapeDtypeStruct((NP, H), jnp.float32),
        jax.ShapeDtypeStruct((NP, H), jnp.float32),
        jax.ShapeDtypeStruct((NP, H), jnp.float32),
        jax.ShapeDtypeStruct((NP, H), jnp.float32),
    ),
)


def _mid_body(aggp_ref, s_ref, dib_ref, dob_ref, b_ref, wg_ref, ws_ref,
              pn_ref, sn_ref):
    agg = aggp_ref[:, 0:H] + aggp_ref[:, H:2 * H]
    h = jnp.maximum(dib_ref[...] * agg + s_ref[...] + b_ref[...], 0.0)
    pn_ref[...] = dob_ref[...] * jnp.dot(h, wg_ref[...],
                                         preferred_element_type=jnp.float32)
    sn_ref[...] = jnp.dot(h, ws_ref[...], preferred_element_type=jnp.float32)


_tc_mid = pl.pallas_call(
    _mid_body,
    out_shape=(
        jax.ShapeDtypeStruct((NP, H), jnp.float32),
        jax.ShapeDtypeStruct((NP, H), jnp.float32),
    ),
)


def _final_body(aggp_ref, s_ref, dib_ref, b_ref, i_ref, wd_ref, bd_ref, out_ref):
    agg = aggp_ref[:, 0:H] + aggp_ref[:, H:2 * H]
    h = jnp.maximum(dib_ref[...] * agg + s_ref[...] + b_ref[...], 0.0)
    gids = lax.broadcasted_iota(jnp.int32, (NG, NP), 0)
    onehot_t = (gids == i_ref[...]).astype(jnp.float32)
    sums = jnp.dot(onehot_t, h, preferred_element_type=jnp.float32)
    counts = jnp.sum(onehot_t, axis=1, keepdims=True)
    pooled = sums / jnp.maximum(counts, 1.0)
    logits = jnp.dot(pooled, wd_ref[...],
                     preferred_element_type=jnp.float32) + bd_ref[...]
    m = jnp.max(logits, axis=1, keepdims=True)
    e = jnp.exp(logits - m)
    out_ref[...] = e / jnp.sum(e, axis=1, keepdims=True)


_tc_final = pl.pallas_call(
    _final_body,
    out_shape=jax.ShapeDtypeStruct((NG, NL), jnp.float32),
)


def kernel(x, edge_index, i, Wg1, Ws1, b1, Wg2, Ws2, b2, Wg3, Ws3, b3, Wd, bd):
    xp = jnp.pad(x, ((0, NP - NN), (0, 0)))
    ip = jnp.pad(i, (0, NP - NN), constant_values=NG).reshape(1, NP)
    z16 = jnp.zeros((NP, 16), jnp.float32)
    z32 = jnp.zeros((NP, H), jnp.float32)
    ones16 = jnp.ones((CH, 16), jnp.float32)

    deg = _sc_deg(edge_index, ones16, z16)
    p1, s1, dib, dob = _tc_prep(xp, deg, Wg1, Ws1)
    agg1 = _sc_agg(p1, edge_index, z32)
    p2, s2 = _tc_mid(agg1, s1, dib, dob, b1.reshape(1, H), Wg2, Ws2)
    agg2 = _sc_agg(p2, edge_index, z32)
    p3, s3 = _tc_mid(agg2, s2, dib, dob, b2.reshape(1, H), Wg3, Ws3)
    agg3 = _sc_agg(p3, edge_index, z32)
    return _tc_final(agg3, s3, dib, b3.reshape(1, H), ip, Wd,
                     bd.reshape(1, NL))


# speedup vs baseline: 1.1433x; 1.1433x over previous
import functools

import jax
import jax.numpy as jnp
from jax import lax
from jax.experimental import pallas as pl
from jax.experimental.pallas import tpu as pltpu
from jax.experimental.pallas import tpu_sc as plsc

NN = 10000
EE = 320000
DF = 128
H = 32
NG = 128
NL = 10

NC = 2
NS = 16
NW = NC * NS
CH = 128
KCH = 79
EP = NW * KCH * CH
NP = NN + 112
RT = NP // NS

_mesh = plsc.VectorSubcoreMesh(core_axis_name="c", subcore_axis_name="s")
_sc_params = pltpu.CompilerParams(use_tc_tiling_on_sc=False)


EPT = EE // NW
EFILL = KCH * CH - EPT


def _stage_edges(e_h, wid, src_v, dst_v):
    pltpu.sync_copy(e_h.at[0, pl.ds(wid * EPT, EPT)], src_v.at[pl.ds(0, EPT)])
    pltpu.sync_copy(e_h.at[1, pl.ds(wid * EPT, EPT)], dst_v.at[pl.ds(0, EPT)])
    pad16 = lax.iota(jnp.int32, 16) + NN

    def fill(g, carry):
        src_v[pl.ds(EPT + g * 16, 16)] = pad16 + g * 16
        dst_v[pl.ds(EPT + g * 16, 16)] = pad16 + g * 16
        return carry

    lax.fori_loop(0, EFILL // 16, fill, 0)


def _deg_body(e_h, ones_h, z_h, deg_h,
              src_v, dst_v, ones_v, acc_in, acc_out, isem, osem):
    c = lax.axis_index("c")
    s = lax.axis_index("s")
    wid = s * NC + c
    pltpu.sync_copy(z_h.at[pl.ds(s * RT, RT)], acc_in.at[pl.ds(s * RT, RT)])
    pltpu.sync_copy(z_h.at[pl.ds(s * RT, RT)], acc_out.at[pl.ds(s * RT, RT)])
    pltpu.sync_copy(ones_h, ones_v)
    _stage_edges(e_h, wid, src_v, dst_v)
    plsc.subcore_barrier()

    def body(j, carry):
        pltpu.async_copy(ones_v, acc_in.at[dst_v.at[pl.ds(j * CH, CH)]],
                         isem, add=True)
        pltpu.async_copy(ones_v, acc_out.at[src_v.at[pl.ds(j * CH, CH)]],
                         osem, add=True)
        return carry

    lax.fori_loop(0, KCH, body, 0)

    def drain(j, carry):
        pltpu.make_async_copy(ones_v, acc_in.at[dst_v.at[pl.ds(j * CH, CH)]],
                              isem).wait()
        pltpu.make_async_copy(ones_v, acc_out.at[src_v.at[pl.ds(j * CH, CH)]],
                              osem).wait()
        return carry

    lax.fori_loop(0, KCH, drain, 0)
    plsc.subcore_barrier()
    pltpu.sync_copy(acc_in.at[pl.ds(s * RT, RT)],
                    deg_h.at[pl.ds(s * RT, RT), pl.ds(c * 16, 16)])
    pltpu.sync_copy(acc_out.at[pl.ds(s * RT, RT)],
                    deg_h.at[pl.ds(s * RT, RT), pl.ds(32 + c * 16, 16)])


_sc_deg = pl.kernel(
    _deg_body,
    out_type=jax.ShapeDtypeStruct((NP, 128), jnp.float32),
    mesh=_mesh,
    scratch_types=[
        pltpu.VMEM((KCH * CH,), jnp.int32),
        pltpu.VMEM((KCH * CH,), jnp.int32),
        pltpu.VMEM((CH, 16), jnp.float32),
        pltpu.VMEM_SHARED((NP, 16), jnp.float32),
        pltpu.VMEM_SHARED((NP, 16), jnp.float32),
        pltpu.SemaphoreType.DMA,
        pltpu.SemaphoreType.DMA,
    ],
    compiler_params=_sc_params,
)


NBUF = 4


def _agg_body(p_h, e_h, z_h, out_h,
              src_v, dst_v, rows, p_sh, acc, gsems, ssems):
    c = lax.axis_index("c")
    s = lax.axis_index("s")
    wid = s * NC + c
    pltpu.sync_copy(z_h.at[pl.ds(s * RT, RT)], acc.at[pl.ds(s * RT, RT)])
    pltpu.sync_copy(p_h.at[pl.ds(s * RT, RT), pl.ds(0, H)],
                    p_sh.at[pl.ds(s * RT, RT)])
    _stage_edges(e_h, wid, src_v, dst_v)
    plsc.subcore_barrier()

    def gsrc(j):
        return p_sh.at[src_v.at[pl.ds(j * CH, CH)]]

    def sidx(j):
        return dst_v.at[pl.ds(j * CH, CH)]

    def step(j, b):
        buf = rows.at[b]
        pltpu.make_async_copy(gsrc(j), buf, gsems.at[b]).wait()
        pltpu.async_copy(buf, acc.at[sidx(j)], ssems.at[b], add=True)

        @pl.when(j + 2 < KCH)
        def _():
            b2 = (b + 2) % NBUF
            buf2 = rows.at[b2]

            @pl.when(j >= 2)
            def _():
                pltpu.make_async_copy(
                    buf2, acc.at[sidx(j - 2)], ssems.at[b2]).wait()

            pltpu.async_copy(gsrc(j + 2), buf2, gsems.at[b2])

    pltpu.async_copy(gsrc(0), rows.at[0], gsems.at[0])
    pltpu.async_copy(gsrc(1), rows.at[1], gsems.at[1])

    def body(j, carry):
        for b in range(NBUF):
            @pl.when(lax.rem(j, NBUF) == b)
            def _(j=j, b=b):
                step(j, b)
        return carry

    lax.fori_loop(0, KCH, body, 0)
    for t in range(KCH - NBUF, KCH):
        b = t % NBUF
        pltpu.make_async_copy(rows.at[b], acc.at[sidx(t)], ssems.at[b]).wait()
    plsc.subcore_barrier()
    pltpu.sync_copy(acc.at[pl.ds(s * RT, RT)],
                    out_h.at[pl.ds(s * RT, RT), pl.ds(c * H, H)])


_sc_agg = pl.kernel(
    _agg_body,
    out_type=jax.ShapeDtypeStruct((NP, 128), jnp.float32),
    mesh=_mesh,
    scratch_types=[
        pltpu.VMEM((KCH * CH,), jnp.int32),
        pltpu.VMEM((KCH * CH,), jnp.int32),
        pltpu.VMEM((NBUF, CH, H), jnp.float32),
        pltpu.VMEM_SHARED((NP, H), jnp.float32),
        pltpu.VMEM_SHARED((NP, H), jnp.float32),
        pltpu.SemaphoreType.DMA((NBUF,)),
        pltpu.SemaphoreType.DMA((NBUF,)),
    ],
    compiler_params=_sc_params,
)


def _prep_body(x_ref, deg_ref, wg_ref, ws_ref,
               p_ref, s_ref, dib_ref, dob_ref):
    deg_in = deg_ref[:, 0:1] + deg_ref[:, 16:17]
    deg_out = deg_ref[:, 32:33] + deg_ref[:, 48:49]
    di = jnp.where(deg_in > 0, lax.rsqrt(jnp.maximum(deg_in, 1.0)), 0.0)
    do = jnp.where(deg_out > 0, lax.rsqrt(jnp.maximum(deg_out, 1.0)), 0.0)
    x = x_ref[...]
    p_ref[:, 0:H] = do * jnp.dot(x, wg_ref[...], preferred_element_type=jnp.float32)
    s_ref[...] = jnp.dot(x, ws_ref[...], preferred_element_type=jnp.float32)
    dib_ref[...] = jnp.broadcast_to(di, (NP, H))
    dob_ref[...] = jnp.broadcast_to(do, (NP, H))


_tc_prep = pl.pallas_call(
    _prep_body,
    out_shape=(
        jax.ShapeDtypeStruct((NP, 128), jnp.float32),
        jax.ShapeDtypeStruct((NP, H), jnp.float32),
        jax.ShapeDtypeStruct((NP, H), jnp.float32),
        jax.ShapeDtypeStruct((NP, H), jnp.float32),
    ),
)


def _mid_body(aggp_ref, s_ref, dib_ref, dob_ref, b_ref, wg_ref, ws_ref,
              pn_ref, sn_ref):
    agg = aggp_ref[:, 0:H] + aggp_ref[:, H:2 * H]
    h = jnp.maximum(dib_ref[...] * agg + s_ref[...] + b_ref[...], 0.0)
    pn_ref[:, 0:H] = dob_ref[...] * jnp.dot(h, wg_ref[...],
                                            preferred_element_type=jnp.float32)
    sn_ref[...] = jnp.dot(h, ws_ref[...], preferred_element_type=jnp.float32)


_tc_mid = pl.pallas_call(
    _mid_body,
    out_shape=(
        jax.ShapeDtypeStruct((NP, 128), jnp.float32),
        jax.ShapeDtypeStruct((NP, H), jnp.float32),
    ),
)


def _final_body(aggp_ref, s_ref, dib_ref, b_ref, i_ref, wd_ref, bd_ref, out_ref):
    agg = aggp_ref[:, 0:H] + aggp_ref[:, H:2 * H]
    h = jnp.maximum(dib_ref[...] * agg + s_ref[...] + b_ref[...], 0.0)
    gids = lax.broadcasted_iota(jnp.int32, (NG, NP), 0)
    onehot_t = (gids == i_ref[...]).astype(jnp.float32)
    sums = jnp.dot(onehot_t, h, preferred_element_type=jnp.float32)
    counts = jnp.sum(onehot_t, axis=1, keepdims=True)
    pooled = sums / jnp.maximum(counts, 1.0)
    logits = jnp.dot(pooled, wd_ref[...],
                     preferred_element_type=jnp.float32) + bd_ref[...]
    m = jnp.max(logits, axis=1, keepdims=True)
    e = jnp.exp(logits - m)
    out_ref[...] = e / jnp.sum(e, axis=1, keepdims=True)


_tc_final = pl.pallas_call(
    _final_body,
    out_shape=jax.ShapeDtypeStruct((NG, NL), jnp.float32),
)


def kernel(x, edge_index, i, Wg1, Ws1, b1, Wg2, Ws2, b2, Wg3, Ws3, b3, Wd, bd):
    xp = jnp.pad(x, ((0, NP - NN), (0, 0)))
    ip = jnp.pad(i, (0, NP - NN), constant_values=NG).reshape(1, NP)
    z16 = jnp.zeros((NP, 16), jnp.float32)
    z32 = jnp.zeros((NP, H), jnp.float32)
    ones16 = jnp.ones((CH, 16), jnp.float32)

    deg = _sc_deg(edge_index, ones16, z16)
    p1, s1, dib, dob = _tc_prep(xp, deg, Wg1, Ws1)
    agg1 = _sc_agg(p1, edge_index, z32)
    p2, s2 = _tc_mid(agg1, s1, dib, dob, b1.reshape(1, H), Wg2, Ws2)
    agg2 = _sc_agg(p2, edge_index, z32)
    p3, s3 = _tc_mid(agg2, s2, dib, dob, b2.reshape(1, H), Wg3, Ws3)
    agg3 = _sc_agg(p3, edge_index, z32)
    return _tc_final(agg3, s3, dib, b3.reshape(1, H), ip, Wd,
                     bd.reshape(1, NL))


# speedup vs baseline: 1.2250x; 1.0715x over previous
import functools

import jax
import jax.numpy as jnp
from jax import lax
from jax.experimental import pallas as pl
from jax.experimental.pallas import tpu as pltpu
from jax.experimental.pallas import tpu_sc as plsc

NN = 10000
EE = 320000
DF = 128
H = 32
NG = 128
NL = 10

NC = 2
NS = 16
NW = NC * NS
CH = 128
KCH = 79
EP = NW * KCH * CH
NP = NN + 112
RT = NP // NS

_mesh = plsc.VectorSubcoreMesh(core_axis_name="c", subcore_axis_name="s")
_sc_params = pltpu.CompilerParams(use_tc_tiling_on_sc=False)


EPT = EE // NW
EFILL = KCH * CH - EPT


def _stage_edges_start(e_h, wid, src_v, dst_v, sem):
    pltpu.async_copy(e_h.at[0, pl.ds(wid * EPT, EPT)], src_v.at[pl.ds(0, EPT)], sem)
    pltpu.async_copy(e_h.at[1, pl.ds(wid * EPT, EPT)], dst_v.at[pl.ds(0, EPT)], sem)


def _stage_edges_finish(e_h, wid, src_v, dst_v, sem):
    pltpu.make_async_copy(
        e_h.at[0, pl.ds(wid * EPT, EPT)], src_v.at[pl.ds(0, EPT)], sem).wait()
    pltpu.make_async_copy(
        e_h.at[1, pl.ds(wid * EPT, EPT)], dst_v.at[pl.ds(0, EPT)], sem).wait()
    pad16 = lax.iota(jnp.int32, 16) + NN

    def fill(g, carry):
        src_v[pl.ds(EPT + g * 16, 16)] = pad16 + g * 16
        dst_v[pl.ds(EPT + g * 16, 16)] = pad16 + g * 16
        return carry

    lax.fori_loop(0, EFILL // 16, fill, 0)


def _deg_body(e_h, ones_h, z_h, deg_h,
              src_v, dst_v, ones_v, acc_in, acc_out, isem, osem):
    c = lax.axis_index("c")
    s = lax.axis_index("s")
    wid = s * NC + c
    _stage_edges_start(e_h, wid, src_v, dst_v, isem)
    pltpu.async_copy(z_h.at[pl.ds(s * RT, RT)], acc_in.at[pl.ds(s * RT, RT)], osem)
    pltpu.async_copy(z_h.at[pl.ds(s * RT, RT)], acc_out.at[pl.ds(s * RT, RT)], osem)
    pltpu.sync_copy(ones_h, ones_v)
    _stage_edges_finish(e_h, wid, src_v, dst_v, isem)
    pltpu.make_async_copy(z_h.at[pl.ds(s * RT, RT)],
                          acc_in.at[pl.ds(s * RT, RT)], osem).wait()
    pltpu.make_async_copy(z_h.at[pl.ds(s * RT, RT)],
                          acc_out.at[pl.ds(s * RT, RT)], osem).wait()
    plsc.subcore_barrier()

    def body(j, carry):
        pltpu.async_copy(ones_v, acc_in.at[dst_v.at[pl.ds(j * CH, CH)]],
                         isem, add=True)
        pltpu.async_copy(ones_v, acc_out.at[src_v.at[pl.ds(j * CH, CH)]],
                         osem, add=True)
        return carry

    lax.fori_loop(0, KCH, body, 0)

    def drain(j, carry):
        pltpu.make_async_copy(ones_v, acc_in.at[dst_v.at[pl.ds(j * CH, CH)]],
                              isem).wait()
        pltpu.make_async_copy(ones_v, acc_out.at[src_v.at[pl.ds(j * CH, CH)]],
                              osem).wait()
        return carry

    lax.fori_loop(0, KCH, drain, 0)
    plsc.subcore_barrier()
    pltpu.sync_copy(acc_in.at[pl.ds(s * RT, RT)],
                    deg_h.at[pl.ds(s * RT, RT), pl.ds(c * 8, 8)])
    pltpu.sync_copy(acc_out.at[pl.ds(s * RT, RT)],
                    deg_h.at[pl.ds(s * RT, RT), pl.ds(16 + c * 8, 8)])


_sc_deg = pl.kernel(
    _deg_body,
    out_type=jax.ShapeDtypeStruct((NP, 128), jnp.float32),
    mesh=_mesh,
    scratch_types=[
        pltpu.VMEM((KCH * CH,), jnp.int32),
        pltpu.VMEM((KCH * CH,), jnp.int32),
        pltpu.VMEM((CH, 8), jnp.float32),
        pltpu.VMEM_SHARED((NP, 8), jnp.float32),
        pltpu.VMEM_SHARED((NP, 8), jnp.float32),
        pltpu.SemaphoreType.DMA,
        pltpu.SemaphoreType.DMA,
    ],
    compiler_params=_sc_params,
)


NBUF = 4


def _agg_body(p_h, e_h, z_h, out_h,
              src_v, dst_v, rows, p_sh, acc, gsems, ssems):
    c = lax.axis_index("c")
    s = lax.axis_index("s")
    wid = s * NC + c
    _stage_edges_start(e_h, wid, src_v, dst_v, gsems.at[0])
    pltpu.async_copy(z_h.at[pl.ds(s * RT, RT)], acc.at[pl.ds(s * RT, RT)],
                     ssems.at[0])
    pltpu.async_copy(p_h.at[pl.ds(s * RT, RT), pl.ds(0, H)],
                     p_sh.at[pl.ds(s * RT, RT)], ssems.at[1])
    _stage_edges_finish(e_h, wid, src_v, dst_v, gsems.at[0])
    pltpu.make_async_copy(z_h.at[pl.ds(s * RT, RT)],
                          acc.at[pl.ds(s * RT, RT)], ssems.at[0]).wait()
    pltpu.make_async_copy(p_h.at[pl.ds(s * RT, RT), pl.ds(0, H)],
                          p_sh.at[pl.ds(s * RT, RT)], ssems.at[1]).wait()
    plsc.subcore_barrier()

    def gsrc(j):
        return p_sh.at[src_v.at[pl.ds(j * CH, CH)]]

    def sidx(j):
        return dst_v.at[pl.ds(j * CH, CH)]

    def step(j, b):
        buf = rows.at[b]
        pltpu.make_async_copy(gsrc(j), buf, gsems.at[b]).wait()
        pltpu.async_copy(buf, acc.at[sidx(j)], ssems.at[b], add=True)

        @pl.when(j + 2 < KCH)
        def _():
            b2 = (b + 2) % NBUF
            buf2 = rows.at[b2]

            @pl.when(j >= 2)
            def _():
                pltpu.make_async_copy(
                    buf2, acc.at[sidx(j - 2)], ssems.at[b2]).wait()

            pltpu.async_copy(gsrc(j + 2), buf2, gsems.at[b2])

    pltpu.async_copy(gsrc(0), rows.at[0], gsems.at[0])
    pltpu.async_copy(gsrc(1), rows.at[1], gsems.at[1])

    def body(j, carry):
        for b in range(NBUF):
            @pl.when(lax.rem(j, NBUF) == b)
            def _(j=j, b=b):
                step(j, b)
        return carry

    lax.fori_loop(0, KCH, body, 0)
    for t in range(KCH - NBUF, KCH):
        b = t % NBUF
        pltpu.make_async_copy(rows.at[b], acc.at[sidx(t)], ssems.at[b]).wait()
    plsc.subcore_barrier()
    pltpu.sync_copy(acc.at[pl.ds(s * RT, RT)],
                    out_h.at[pl.ds(s * RT, RT), pl.ds(c * H, H)])


_sc_agg = pl.kernel(
    _agg_body,
    out_type=jax.ShapeDtypeStruct((NP, 128), jnp.float32),
    mesh=_mesh,
    scratch_types=[
        pltpu.VMEM((KCH * CH,), jnp.int32),
        pltpu.VMEM((KCH * CH,), jnp.int32),
        pltpu.VMEM((NBUF, CH, H), jnp.float32),
        pltpu.VMEM_SHARED((NP, H), jnp.float32),
        pltpu.VMEM_SHARED((NP, H), jnp.float32),
        pltpu.SemaphoreType.DMA((NBUF,)),
        pltpu.SemaphoreType.DMA((NBUF,)),
    ],
    compiler_params=_sc_params,
)


def _prep_body(x_ref, deg_ref, wg_ref, ws_ref,
               p_ref, s_ref, dib_ref, dob_ref):
    deg_in = deg_ref[:, 0:1] + deg_ref[:, 8:9]
    deg_out = deg_ref[:, 16:17] + deg_ref[:, 24:25]
    di = jnp.where(deg_in > 0, lax.rsqrt(jnp.maximum(deg_in, 1.0)), 0.0)
    do = jnp.where(deg_out > 0, lax.rsqrt(jnp.maximum(deg_out, 1.0)), 0.0)
    x = x_ref[...]
    p_ref[:, 0:H] = do * jnp.dot(x, wg_ref[...], preferred_element_type=jnp.float32)
    s_ref[...] = jnp.dot(x, ws_ref[...], preferred_element_type=jnp.float32)
    dib_ref[...] = jnp.broadcast_to(di, (NP, H))
    dob_ref[...] = jnp.broadcast_to(do, (NP, H))


_tc_prep = pl.pallas_call(
    _prep_body,
    out_shape=(
        jax.ShapeDtypeStruct((NP, 128), jnp.float32),
        jax.ShapeDtypeStruct((NP, H), jnp.float32),
        jax.ShapeDtypeStruct((NP, H), jnp.float32),
        jax.ShapeDtypeStruct((NP, H), jnp.float32),
    ),
)


def _mid_body(aggp_ref, s_ref, dib_ref, dob_ref, b_ref, wg_ref, ws_ref,
              pn_ref, sn_ref):
    agg = aggp_ref[:, 0:H] + aggp_ref[:, H:2 * H]
    h = jnp.maximum(dib_ref[...] * agg + s_ref[...] + b_ref[...], 0.0)
    pn_ref[:, 0:H] = dob_ref[...] * jnp.dot(h, wg_ref[...],
                                            preferred_element_type=jnp.float32)
    sn_ref[...] = jnp.dot(h, ws_ref[...], preferred_element_type=jnp.float32)


_tc_mid = pl.pallas_call(
    _mid_body,
    out_shape=(
        jax.ShapeDtypeStruct((NP, 128), jnp.float32),
        jax.ShapeDtypeStruct((NP, H), jnp.float32),
    ),
)


def _final_body(aggp_ref, s_ref, dib_ref, b_ref, i_ref, wd_ref, bd_ref, out_ref):
    agg = aggp_ref[:, 0:H] + aggp_ref[:, H:2 * H]
    h = jnp.maximum(dib_ref[...] * agg + s_ref[...] + b_ref[...], 0.0)
    gids = lax.broadcasted_iota(jnp.int32, (NG, NP), 0)
    onehot_t = (gids == i_ref[...]).astype(jnp.float32)
    sums = jnp.dot(onehot_t, h, preferred_element_type=jnp.float32)
    counts = jnp.sum(onehot_t, axis=1, keepdims=True)
    pooled = sums / jnp.maximum(counts, 1.0)
    logits = jnp.dot(pooled, wd_ref[...],
                     preferred_element_type=jnp.float32) + bd_ref[...]
    m = jnp.max(logits, axis=1, keepdims=True)
    e = jnp.exp(logits - m)
    out_ref[...] = e / jnp.sum(e, axis=1, keepdims=True)


_tc_final = pl.pallas_call(
    _final_body,
    out_shape=jax.ShapeDtypeStruct((NG, NL), jnp.float32),
)


def kernel(x, edge_index, i, Wg1, Ws1, b1, Wg2, Ws2, b2, Wg3, Ws3, b3, Wd, bd):
    xp = jnp.pad(x, ((0, NP - NN), (0, 0)))
    ip = jnp.pad(i, (0, NP - NN), constant_values=NG).reshape(1, NP)
    z8 = jnp.zeros((NP, 8), jnp.float32)
    z32 = jnp.zeros((NP, H), jnp.float32)
    ones8 = jnp.ones((CH, 8), jnp.float32)

    deg = _sc_deg(edge_index, ones8, z8)
    p1, s1, dib, dob = _tc_prep(xp, deg, Wg1, Ws1)
    agg1 = _sc_agg(p1, edge_index, z32)
    p2, s2 = _tc_mid(agg1, s1, dib, dob, b1.reshape(1, H), Wg2, Ws2)
    agg2 = _sc_agg(p2, edge_index, z32)
    p3, s3 = _tc_mid(agg2, s2, dib, dob, b2.reshape(1, H), Wg3, Ws3)
    agg3 = _sc_agg(p3, edge_index, z32)
    return _tc_final(agg3, s3, dib, b3.reshape(1, H), ip, Wd,
                     bd.reshape(1, NL))


# speedup vs baseline: 1.2328x; 1.0064x over previous
import functools

import jax
import jax.numpy as jnp
from jax import lax
from jax.experimental import pallas as pl
from jax.experimental.pallas import tpu as pltpu
from jax.experimental.pallas import tpu_sc as plsc

NN = 10000
EE = 320000
DF = 128
H = 32
NG = 128
NL = 10

NC = 2
NS = 16
NW = NC * NS
CH = 128
KCH = 79
EP = NW * KCH * CH
NP = NN + 112
RT = NP // NS

_mesh = plsc.VectorSubcoreMesh(core_axis_name="c", subcore_axis_name="s")
_sc_params = pltpu.CompilerParams(use_tc_tiling_on_sc=False)


EPT = EE // NW
EFILL = KCH * CH - EPT


def _stage_edges_start(e_h, wid, src_v, dst_v, sem):
    pltpu.async_copy(e_h.at[0, pl.ds(wid * EPT, EPT)], src_v.at[pl.ds(0, EPT)], sem)
    pltpu.async_copy(e_h.at[1, pl.ds(wid * EPT, EPT)], dst_v.at[pl.ds(0, EPT)], sem)


def _stage_edges_finish(e_h, wid, src_v, dst_v, sem):
    pltpu.make_async_copy(
        e_h.at[0, pl.ds(wid * EPT, EPT)], src_v.at[pl.ds(0, EPT)], sem).wait()
    pltpu.make_async_copy(
        e_h.at[1, pl.ds(wid * EPT, EPT)], dst_v.at[pl.ds(0, EPT)], sem).wait()
    pad16 = lax.iota(jnp.int32, 16) + NN

    def fill(g, carry):
        src_v[pl.ds(EPT + g * 16, 16)] = pad16 + g * 16
        dst_v[pl.ds(EPT + g * 16, 16)] = pad16 + g * 16
        return carry

    lax.fori_loop(0, EFILL // 16, fill, 0)


def _deg_body(e_h, ones_h, z_h, deg_h,
              src_v, dst_v, ones_v, acc_in, acc_out, isem, osem):
    c = lax.axis_index("c")
    s = lax.axis_index("s")
    wid = s * NC + c
    _stage_edges_start(e_h, wid, src_v, dst_v, isem)
    pltpu.async_copy(z_h.at[pl.ds(s * RT, RT)], acc_in.at[pl.ds(s * RT, RT)], osem)
    pltpu.async_copy(z_h.at[pl.ds(s * RT, RT)], acc_out.at[pl.ds(s * RT, RT)], osem)
    pltpu.sync_copy(ones_h, ones_v)
    _stage_edges_finish(e_h, wid, src_v, dst_v, isem)
    pltpu.make_async_copy(z_h.at[pl.ds(s * RT, RT)],
                          acc_in.at[pl.ds(s * RT, RT)], osem).wait()
    pltpu.make_async_copy(z_h.at[pl.ds(s * RT, RT)],
                          acc_out.at[pl.ds(s * RT, RT)], osem).wait()
    plsc.subcore_barrier()

    def body(j, carry):
        pltpu.async_copy(ones_v, acc_in.at[dst_v.at[pl.ds(j * CH, CH)]],
                         isem, add=True)
        pltpu.async_copy(ones_v, acc_out.at[src_v.at[pl.ds(j * CH, CH)]],
                         osem, add=True)
        return carry

    lax.fori_loop(0, KCH, body, 0)

    def drain(j, carry):
        pltpu.make_async_copy(ones_v, acc_in.at[dst_v.at[pl.ds(j * CH, CH)]],
                              isem).wait()
        pltpu.make_async_copy(ones_v, acc_out.at[src_v.at[pl.ds(j * CH, CH)]],
                              osem).wait()
        return carry

    lax.fori_loop(0, KCH, drain, 0)
    plsc.subcore_barrier()
    pltpu.sync_copy(acc_in.at[pl.ds(s * RT, RT)],
                    deg_h.at[pl.ds(s * RT, RT), pl.ds(c * 8, 8)])
    pltpu.sync_copy(acc_out.at[pl.ds(s * RT, RT)],
                    deg_h.at[pl.ds(s * RT, RT), pl.ds(16 + c * 8, 8)])


_sc_deg = pl.kernel(
    _deg_body,
    out_type=jax.ShapeDtypeStruct((NP, 128), jnp.float32),
    mesh=_mesh,
    scratch_types=[
        pltpu.VMEM((KCH * CH,), jnp.int32),
        pltpu.VMEM((KCH * CH,), jnp.int32),
        pltpu.VMEM((CH, 8), jnp.float32),
        pltpu.VMEM_SHARED((NP, 8), jnp.float32),
        pltpu.VMEM_SHARED((NP, 8), jnp.float32),
        pltpu.SemaphoreType.DMA,
        pltpu.SemaphoreType.DMA,
    ],
    compiler_params=_sc_params,
)


NBUF = 4


def _agg_body(p_h, e_h, z_h, out_h,
              src_v, dst_v, rows, p_sh, acc, gsems, ssems):
    c = lax.axis_index("c")
    s = lax.axis_index("s")
    wid = s * NC + c
    _stage_edges_start(e_h, wid, src_v, dst_v, gsems.at[0])
    pltpu.async_copy(z_h.at[pl.ds(s * RT, RT)], acc.at[pl.ds(s * RT, RT)],
                     ssems.at[0])
    pltpu.async_copy(p_h.at[pl.ds(s * RT, RT), pl.ds(0, H)],
                     p_sh.at[pl.ds(s * RT, RT)], ssems.at[1])
    _stage_edges_finish(e_h, wid, src_v, dst_v, gsems.at[0])
    pltpu.make_async_copy(z_h.at[pl.ds(s * RT, RT)],
                          acc.at[pl.ds(s * RT, RT)], ssems.at[0]).wait()
    pltpu.make_async_copy(p_h.at[pl.ds(s * RT, RT), pl.ds(0, H)],
                          p_sh.at[pl.ds(s * RT, RT)], ssems.at[1]).wait()
    plsc.subcore_barrier()

    def gsrc(j):
        return p_sh.at[src_v.at[pl.ds(j * CH, CH)]]

    def sidx(j):
        return dst_v.at[pl.ds(j * CH, CH)]

    def step(j, b):
        buf = rows.at[b]
        pltpu.make_async_copy(gsrc(j), buf, gsems.at[b]).wait()
        pltpu.async_copy(buf, acc.at[sidx(j)], ssems.at[b], add=True)

        @pl.when(j + 2 < KCH)
        def _():
            b2 = (b + 2) % NBUF
            buf2 = rows.at[b2]

            @pl.when(j >= 2)
            def _():
                pltpu.make_async_copy(
                    buf2, acc.at[sidx(j - 2)], ssems.at[b2]).wait()

            pltpu.async_copy(gsrc(j + 2), buf2, gsems.at[b2])

    pltpu.async_copy(gsrc(0), rows.at[0], gsems.at[0])
    pltpu.async_copy(gsrc(1), rows.at[1], gsems.at[1])

    def body(j, carry):
        for b in range(NBUF):
            @pl.when(lax.rem(j, NBUF) == b)
            def _(j=j, b=b):
                step(j, b)
        return carry

    lax.fori_loop(0, KCH, body, 0)
    for t in range(KCH - NBUF, KCH):
        b = t % NBUF
        pltpu.make_async_copy(rows.at[b], acc.at[sidx(t)], ssems.at[b]).wait()
    plsc.subcore_barrier()
    pltpu.sync_copy(acc.at[pl.ds(s * RT, RT)],
                    out_h.at[pl.ds(s * RT, RT), pl.ds(c * H, H)])


_sc_agg = pl.kernel(
    _agg_body,
    out_type=jax.ShapeDtypeStruct((NP, 128), jnp.float32),
    mesh=_mesh,
    scratch_types=[
        pltpu.VMEM((KCH * CH,), jnp.int32),
        pltpu.VMEM((KCH * CH,), jnp.int32),
        pltpu.VMEM((NBUF, CH, H), jnp.float32),
        pltpu.VMEM_SHARED((NP, H), jnp.float32),
        pltpu.VMEM_SHARED((NP, H), jnp.float32),
        pltpu.SemaphoreType.DMA((NBUF,)),
        pltpu.SemaphoreType.DMA((NBUF,)),
    ],
    compiler_params=_sc_params,
)


def _mm1_body(x_ref, wg_ref, ws_ref, q_ref, s_ref):
    x = x_ref[...]
    q_ref[...] = jnp.dot(x, wg_ref[...], preferred_element_type=jnp.float32)
    s_ref[...] = jnp.dot(x, ws_ref[...], preferred_element_type=jnp.float32)


_tc_mm1 = pl.pallas_call(
    _mm1_body,
    out_shape=(
        jax.ShapeDtypeStruct((NP, H), jnp.float32),
        jax.ShapeDtypeStruct((NP, H), jnp.float32),
    ),
)


def _scale_body(deg_ref, q_ref, p_ref, dib_ref, dob_ref):
    deg_in = deg_ref[:, 0:1] + deg_ref[:, 8:9]
    deg_out = deg_ref[:, 16:17] + deg_ref[:, 24:25]
    di = jnp.where(deg_in > 0, lax.rsqrt(jnp.maximum(deg_in, 1.0)), 0.0)
    do = jnp.where(deg_out > 0, lax.rsqrt(jnp.maximum(deg_out, 1.0)), 0.0)
    p_ref[:, 0:H] = do * q_ref[...]
    dib_ref[...] = jnp.broadcast_to(di, (NP, H))
    dob_ref[...] = jnp.broadcast_to(do, (NP, H))


_tc_scale = pl.pallas_call(
    _scale_body,
    out_shape=(
        jax.ShapeDtypeStruct((NP, 128), jnp.float32),
        jax.ShapeDtypeStruct((NP, H), jnp.float32),
        jax.ShapeDtypeStruct((NP, H), jnp.float32),
    ),
)


def _mid_body(aggp_ref, s_ref, dib_ref, dob_ref, b_ref, wg_ref, ws_ref,
              pn_ref, sn_ref):
    agg = aggp_ref[:, 0:H] + aggp_ref[:, H:2 * H]
    h = jnp.maximum(dib_ref[...] * agg + s_ref[...] + b_ref[...], 0.0)
    pn_ref[:, 0:H] = dob_ref[...] * jnp.dot(h, wg_ref[...],
                                            preferred_element_type=jnp.float32)
    sn_ref[...] = jnp.dot(h, ws_ref[...], preferred_element_type=jnp.float32)


_tc_mid = pl.pallas_call(
    _mid_body,
    out_shape=(
        jax.ShapeDtypeStruct((NP, 128), jnp.float32),
        jax.ShapeDtypeStruct((NP, H), jnp.float32),
    ),
)


def _final_body(aggp_ref, s_ref, dib_ref, b_ref, i_ref, wd_ref, bd_ref, out_ref):
    agg = aggp_ref[:, 0:H] + aggp_ref[:, H:2 * H]
    h = jnp.maximum(dib_ref[...] * agg + s_ref[...] + b_ref[...], 0.0)
    gids = lax.broadcasted_iota(jnp.int32, (NG, NP), 0)
    onehot_t = (gids == i_ref[...]).astype(jnp.float32)
    sums = jnp.dot(onehot_t, h, preferred_element_type=jnp.float32)
    counts = jnp.sum(onehot_t, axis=1, keepdims=True)
    pooled = sums / jnp.maximum(counts, 1.0)
    logits = jnp.dot(pooled, wd_ref[...],
                     preferred_element_type=jnp.float32) + bd_ref[...]
    m = jnp.max(logits, axis=1, keepdims=True)
    e = jnp.exp(logits - m)
    out_ref[...] = e / jnp.sum(e, axis=1, keepdims=True)


_tc_final = pl.pallas_call(
    _final_body,
    out_shape=jax.ShapeDtypeStruct((NG, NL), jnp.float32),
)


def kernel(x, edge_index, i, Wg1, Ws1, b1, Wg2, Ws2, b2, Wg3, Ws3, b3, Wd, bd):
    xp = jnp.pad(x, ((0, NP - NN), (0, 0)))
    ip = jnp.pad(i, (0, NP - NN), constant_values=NG).reshape(1, NP)
    z8 = jnp.zeros((NP, 8), jnp.float32)
    z32 = jnp.zeros((NP, H), jnp.float32)
    ones8 = jnp.ones((CH, 8), jnp.float32)

    deg = _sc_deg(edge_index, ones8, z8)
    q1, s1 = _tc_mm1(xp, Wg1, Ws1)
    p1, dib, dob = _tc_scale(deg, q1)
    agg1 = _sc_agg(p1, edge_index, z32)
    p2, s2 = _tc_mid(agg1, s1, dib, dob, b1.reshape(1, H), Wg2, Ws2)
    agg2 = _sc_agg(p2, edge_index, z32)
    p3, s3 = _tc_mid(agg2, s2, dib, dob, b2.reshape(1, H), Wg3, Ws3)
    agg3 = _sc_agg(p3, edge_index, z32)
    return _tc_final(agg3, s3, dib, b3.reshape(1, H), ip, Wd,
                     bd.reshape(1, NL))


# speedup vs baseline: 1.2345x; 1.0014x over previous
import jax
import jax.numpy as jnp
from jax import lax
from jax.experimental import pallas as pl
from jax.experimental.pallas import tpu as pltpu
from jax.experimental.pallas import tpu_sc as plsc

NN = 10000
EE = 320000
DF = 128
H = 32
NG = 128
NL = 10

NC = 2
NS = 16
NW = NC * NS
CH = 128
KCH = 79
EP = NW * KCH * CH
NP = NN + 112
RT = NP // NS

_mesh = plsc.VectorSubcoreMesh(core_axis_name="c", subcore_axis_name="s")
_sc_params = pltpu.CompilerParams(use_tc_tiling_on_sc=False)


EPT = EE // NW
EFILL = KCH * CH - EPT


def _stage_edges_start(e_h, wid, src_v, dst_v, sem):
    pltpu.async_copy(e_h.at[0, pl.ds(wid * EPT, EPT)], src_v.at[pl.ds(0, EPT)], sem)
    pltpu.async_copy(e_h.at[1, pl.ds(wid * EPT, EPT)], dst_v.at[pl.ds(0, EPT)], sem)


def _stage_edges_finish(e_h, wid, src_v, dst_v, sem):
    pltpu.make_async_copy(
        e_h.at[0, pl.ds(wid * EPT, EPT)], src_v.at[pl.ds(0, EPT)], sem).wait()
    pltpu.make_async_copy(
        e_h.at[1, pl.ds(wid * EPT, EPT)], dst_v.at[pl.ds(0, EPT)], sem).wait()
    pad16 = lax.iota(jnp.int32, 16) + NN

    def fill(g, carry):
        src_v[pl.ds(EPT + g * 16, 16)] = pad16 + g * 16
        dst_v[pl.ds(EPT + g * 16, 16)] = pad16 + g * 16
        return carry

    lax.fori_loop(0, EFILL // 16, fill, 0)


def _deg_body(e_h, ones_h, z_h, deg_h,
              src_v, dst_v, ones_v, acc_in, acc_out, isem, osem):
    c = lax.axis_index("c")
    s = lax.axis_index("s")
    wid = s * NC + c
    _stage_edges_start(e_h, wid, src_v, dst_v, isem)
    pltpu.async_copy(z_h.at[pl.ds(s * RT, RT)], acc_in.at[pl.ds(s * RT, RT)], osem)
    pltpu.async_copy(z_h.at[pl.ds(s * RT, RT)], acc_out.at[pl.ds(s * RT, RT)], osem)
    pltpu.sync_copy(ones_h, ones_v)
    _stage_edges_finish(e_h, wid, src_v, dst_v, isem)
    pltpu.make_async_copy(z_h.at[pl.ds(s * RT, RT)],
                          acc_in.at[pl.ds(s * RT, RT)], osem).wait()
    pltpu.make_async_copy(z_h.at[pl.ds(s * RT, RT)],
                          acc_out.at[pl.ds(s * RT, RT)], osem).wait()
    plsc.subcore_barrier()

    def body(j, carry):
        pltpu.async_copy(ones_v, acc_in.at[dst_v.at[pl.ds(j * CH, CH)]],
                         isem, add=True)
        pltpu.async_copy(ones_v, acc_out.at[src_v.at[pl.ds(j * CH, CH)]],
                         osem, add=True)
        return carry

    lax.fori_loop(0, KCH, body, 0)

    def drain(j, carry):
        pltpu.make_async_copy(ones_v, acc_in.at[dst_v.at[pl.ds(j * CH, CH)]],
                              isem).wait()
        pltpu.make_async_copy(ones_v, acc_out.at[src_v.at[pl.ds(j * CH, CH)]],
                              osem).wait()
        return carry

    lax.fori_loop(0, KCH, drain, 0)
    plsc.subcore_barrier()
    pltpu.sync_copy(acc_in.at[pl.ds(s * RT, RT)],
                    deg_h.at[pl.ds(s * RT, RT), pl.ds(c * 8, 8)])
    pltpu.sync_copy(acc_out.at[pl.ds(s * RT, RT)],
                    deg_h.at[pl.ds(s * RT, RT), pl.ds(16 + c * 8, 8)])


_sc_deg = pl.kernel(
    _deg_body,
    out_type=jax.ShapeDtypeStruct((NP, 128), jnp.float32),
    mesh=_mesh,
    scratch_types=[
        pltpu.VMEM((KCH * CH,), jnp.int32),
        pltpu.VMEM((KCH * CH,), jnp.int32),
        pltpu.VMEM((CH, 8), jnp.float32),
        pltpu.VMEM_SHARED((NP, 8), jnp.float32),
        pltpu.VMEM_SHARED((NP, 8), jnp.float32),
        pltpu.SemaphoreType.DMA,
        pltpu.SemaphoreType.DMA,
    ],
    compiler_params=_sc_params,
)


NBUF = 4


def _agg_body(p_h, e_h, z_h, out_h,
              src_v, dst_v, rows, p_sh, acc, gsems, ssems):
    c = lax.axis_index("c")
    s = lax.axis_index("s")
    wid = s * NC + c
    _stage_edges_start(e_h, wid, src_v, dst_v, gsems.at[0])
    pltpu.async_copy(z_h.at[pl.ds(s * RT, RT)], acc.at[pl.ds(s * RT, RT)],
                     ssems.at[0])
    pltpu.async_copy(p_h.at[pl.ds(s * RT, RT), pl.ds(0, H)],
                     p_sh.at[pl.ds(s * RT, RT)], ssems.at[1])
    _stage_edges_finish(e_h, wid, src_v, dst_v, gsems.at[0])
    pltpu.make_async_copy(z_h.at[pl.ds(s * RT, RT)],
                          acc.at[pl.ds(s * RT, RT)], ssems.at[0]).wait()
    pltpu.make_async_copy(p_h.at[pl.ds(s * RT, RT), pl.ds(0, H)],
                          p_sh.at[pl.ds(s * RT, RT)], ssems.at[1]).wait()
    plsc.subcore_barrier()

    def gsrc(j):
        return p_sh.at[src_v.at[pl.ds(j * CH, CH)]]

    def sidx(j):
        return dst_v.at[pl.ds(j * CH, CH)]

    def step(j, b):
        buf = rows.at[b]
        pltpu.make_async_copy(gsrc(j), buf, gsems.at[b]).wait()
        pltpu.async_copy(buf, acc.at[sidx(j)], ssems.at[b], add=True)

        @pl.when(j + 2 < KCH)
        def _():
            b2 = (b + 2) % NBUF
            buf2 = rows.at[b2]

            @pl.when(j >= 2)
            def _():
                pltpu.make_async_copy(
                    buf2, acc.at[sidx(j - 2)], ssems.at[b2]).wait()

            pltpu.async_copy(gsrc(j + 2), buf2, gsems.at[b2])

    pltpu.async_copy(gsrc(0), rows.at[0], gsems.at[0])
    pltpu.async_copy(gsrc(1), rows.at[1], gsems.at[1])

    def body(j, carry):
        for b in range(NBUF):
            @pl.when(lax.rem(j, NBUF) == b)
            def _(j=j, b=b):
                step(j, b)
        return carry

    lax.fori_loop(0, KCH, body, 0)
    for t in range(KCH - NBUF, KCH):
        b = t % NBUF
        pltpu.make_async_copy(rows.at[b], acc.at[sidx(t)], ssems.at[b]).wait()
    plsc.subcore_barrier()
    pltpu.sync_copy(acc.at[pl.ds(s * RT, RT)],
                    out_h.at[pl.ds(s * RT, RT), pl.ds(c * H, H)])


_sc_agg = pl.kernel(
    _agg_body,
    out_type=jax.ShapeDtypeStruct((NP, 128), jnp.float32),
    mesh=_mesh,
    scratch_types=[
        pltpu.VMEM((KCH * CH,), jnp.int32),
        pltpu.VMEM((KCH * CH,), jnp.int32),
        pltpu.VMEM((NBUF, CH, H), jnp.float32),
        pltpu.VMEM_SHARED((NP, H), jnp.float32),
        pltpu.VMEM_SHARED((NP, H), jnp.float32),
        pltpu.SemaphoreType.DMA((NBUF,)),
        pltpu.SemaphoreType.DMA((NBUF,)),
    ],
    compiler_params=_sc_params,
)


def _mm1_body(x_ref, wg_ref, ws_ref, q_ref, s_ref):
    x = x_ref[...]
    q_ref[...] = jnp.dot(x, wg_ref[...], preferred_element_type=jnp.float32)
    s_ref[...] = jnp.dot(x, ws_ref[...], preferred_element_type=jnp.float32)


_tc_mm1 = pl.pallas_call(
    _mm1_body,
    out_shape=(
        jax.ShapeDtypeStruct((NP, H), jnp.float32),
        jax.ShapeDtypeStruct((NP, H), jnp.float32),
    ),
)


def _scale_body(deg_ref, q_ref, p_ref, dib_ref, dob_ref):
    deg_in = deg_ref[:, 0:1] + deg_ref[:, 8:9]
    deg_out = deg_ref[:, 16:17] + deg_ref[:, 24:25]
    di = jnp.where(deg_in > 0, lax.rsqrt(jnp.maximum(deg_in, 1.0)), 0.0)
    do = jnp.where(deg_out > 0, lax.rsqrt(jnp.maximum(deg_out, 1.0)), 0.0)
    p_ref[:, 0:H] = do * q_ref[...]
    dib_ref[...] = jnp.broadcast_to(di, (NP, H))
    dob_ref[...] = jnp.broadcast_to(do, (NP, H))


_tc_scale = pl.pallas_call(
    _scale_body,
    out_shape=(
        jax.ShapeDtypeStruct((NP, 128), jnp.float32),
        jax.ShapeDtypeStruct((NP, H), jnp.float32),
        jax.ShapeDtypeStruct((NP, H), jnp.float32),
    ),
)


def _mid_body(aggp_ref, s_ref, dib_ref, dob_ref, b_ref, wg_ref, ws_ref,
              pn_ref, sn_ref):
    agg = aggp_ref[:, 0:H] + aggp_ref[:, H:2 * H]
    h = jnp.maximum(dib_ref[...] * agg + s_ref[...] + b_ref[...], 0.0)
    pn_ref[:, 0:H] = dob_ref[...] * jnp.dot(h, wg_ref[...],
                                            preferred_element_type=jnp.float32)
    sn_ref[...] = jnp.dot(h, ws_ref[...], preferred_element_type=jnp.float32)


_tc_mid = pl.pallas_call(
    _mid_body,
    out_shape=(
        jax.ShapeDtypeStruct((NP, 128), jnp.float32),
        jax.ShapeDtypeStruct((NP, H), jnp.float32),
    ),
)


def _final_body(aggp_ref, s_ref, dib_ref, b_ref, i_ref, wd_ref, bd_ref, out_ref):
    agg = aggp_ref[:, 0:H] + aggp_ref[:, H:2 * H]
    h = jnp.maximum(dib_ref[...] * agg + s_ref[...] + b_ref[...], 0.0)
    gids = lax.broadcasted_iota(jnp.int32, (NG, NP), 0)
    onehot_t = (gids == i_ref[...]).astype(jnp.float32)
    sums = jnp.dot(onehot_t, h, preferred_element_type=jnp.float32)
    counts = jnp.sum(onehot_t, axis=1, keepdims=True)
    pooled = sums / jnp.maximum(counts, 1.0)
    logits = jnp.dot(pooled, wd_ref[...],
                     preferred_element_type=jnp.float32) + bd_ref[...]
    m = jnp.max(logits, axis=1, keepdims=True)
    e = jnp.exp(logits - m)
    out_ref[...] = e / jnp.sum(e, axis=1, keepdims=True)


_tc_final = pl.pallas_call(
    _final_body,
    out_shape=jax.ShapeDtypeStruct((NG, NL), jnp.float32),
)


def kernel(x, edge_index, i, Wg1, Ws1, b1, Wg2, Ws2, b2, Wg3, Ws3, b3, Wd, bd):
    xp = jnp.pad(x, ((0, NP - NN), (0, 0)))
    ip = jnp.pad(i, (0, NP - NN), constant_values=NG).reshape(1, NP)
    z8 = jnp.zeros((NP, 8), jnp.float32)
    z32 = jnp.zeros((NP, H), jnp.float32)
    ones8 = jnp.ones((CH, 8), jnp.float32)

    deg = _sc_deg(edge_index, ones8, z8)
    q1, s1 = _tc_mm1(xp, Wg1, Ws1)
    p1, dib, dob = _tc_scale(deg, q1)
    agg1 = _sc_agg(p1, edge_index, z32)
    p2, s2 = _tc_mid(agg1, s1, dib, dob, b1.reshape(1, H), Wg2, Ws2)
    agg2 = _sc_agg(p2, edge_index, z32)
    p3, s3 = _tc_mid(agg2, s2, dib, dob, b2.reshape(1, H), Wg3, Ws3)
    agg3 = _sc_agg(p3, edge_index, z32)
    return _tc_final(agg3, s3, dib, b3.reshape(1, H), ip, Wd,
                     bd.reshape(1, NL))
